# baseline (speedup 1.0000x reference)
import functools

import jax
import jax.numpy as jnp
from jax import lax
from jax.experimental import pallas as pl
from jax.experimental.pallas import tpu as pltpu
from jax.experimental.pallas import tpu_sc as plsc

_E = 8
_KH = _KW = 3


def _fmt_kernel(x_ref, x1_ref, pool_ref, *, n_rb):
    i = pl.program_id(1)

    @pl.when(i == 0)
    def _():
        pool_ref[...] = jnp.zeros_like(pool_ref)

    @pl.when((i >= 1) & (i <= n_rb))
    def _():
        pool_ref[...] += jnp.sum(x_ref[...], axis=(2, 3))[:, None, :]
        t = jnp.transpose(x_ref[0].astype(jnp.bfloat16), (1, 2, 0))
        x1_ref[0, :, 0 : t.shape[1], :] = t
        x1_ref[0, :, t.shape[1] :, :] = jnp.zeros_like(
            x1_ref[0, :, t.shape[1] :, :])

    @pl.when((i == 0) | (i == n_rb + 1))
    def _():
        x1_ref[...] = jnp.zeros_like(x1_ref)


def _sc_gate(pooled_hbm, gwt_hbm, gb_hbm, ew_hbm, eb_hbm, k_hbm,
             cw_hbm, cb_hbm,
             pooled_v, gw_v, gb_v, k_v, chunk_v, acc_v, eb_v, bb_v, sem,
             *, bsz, chan, n_pixels, rows_per_w):
    nc = 2
    wid = lax.axis_index("s") * nc + lax.axis_index("c")
    iota = lax.broadcasted_iota(jnp.int32, (16,), 0)

    gdn = lax.GatherDimensionNumbers(
        offset_dims=(), collapsed_slice_dims=(0,), start_index_map=(0,))

    def _tree(v, op):
        for sh in (8, 4, 2, 1):
            idx = jnp.bitwise_and(iota + sh, 15)
            rot = lax.gather(v, idx[:, None], gdn, slice_sizes=(1,),
                             mode=lax.GatherScatterMode.PROMISE_IN_BOUNDS)
            v = op(v, rot)
        return v[0]

    def _vsum(v):
        return _tree(v, jnp.add)

    def _vmax(v):
        return _tree(v, jnp.maximum)

    def _vmin(v):
        return _tree(v, jnp.minimum)

    pltpu.sync_copy(pooled_hbm, pooled_v)
    pltpu.sync_copy(gwt_hbm, gw_v)
    pltpu.sync_copy(gb_hbm, gb_v)
    pltpu.sync_copy(k_hbm, k_v)
    pltpu.sync_copy(eb_hbm, eb_v)
    kscal = k_v[...][0]
    gbv = gb_v[...]

    nj = chan // 16
    for b in range(bsz):
        lv = jnp.full((16,), -jnp.inf, jnp.float32)
        for e in range(_E):
            ve = jnp.zeros((16,), jnp.float32)
            for j in range(nj):
                ve = ve + (pooled_v[pl.ds(b * chan + 16 * j, 16)]
                           * gw_v[pl.ds(e * chan + 16 * j, 16)])
            logit = _vsum(ve) * (1.0 / n_pixels) + gbv[e]
            lv = jnp.where(iota == e, logit, lv)
        mask = iota < _E
        mx = _vmax(jnp.where(mask, lv, -jnp.inf))
        ex = jnp.where(mask, jnp.exp(lv - mx), 0.0)
        w = ex / _vsum(ex)
        m1 = _vmax(w)
        i1 = _vmin(jnp.where(w == m1, iota, 16))
        w2 = jnp.where(iota == i1, -1.0, w)
        m2 = _vmax(w2)
        i2 = _vmin(jnp.where(w2 == m2, iota, 16))
        scale = (jnp.where(iota == i1, m1, 0.0)
                 + jnp.where(iota == i2, m2, 0.0)) * kscal

        se = [scale[e] for e in range(_E)]
        n_active = (_KH * _KW * chan) // rows_per_w

        @pl.when(wid < n_active)
        def _():
            cps = []
            for e in range(_E):
                cps.append(pltpu.async_copy(
                    ew_hbm.at[e, pl.ds(wid * rows_per_w, rows_per_w), :],
                    chunk_v.at[e], sem))
            for cp in cps:
                cp.wait()

            def body(r, _):
                for j in range(nj):
                    acc = se[0] * chunk_v[0, r, pl.ds(16 * j, 16)]
                    for e in range(1, _E):
                        acc = acc + se[e] * chunk_v[e, r, pl.ds(16 * j, 16)]
                    acc_v[r, pl.ds(16 * j, 16)] = acc
                return 0

            lax.fori_loop(0, rows_per_w, body, 0)
            pltpu.sync_copy(
                acc_v, cw_hbm.at[b, pl.ds(wid * rows_per_w, rows_per_w), :])

        @pl.when(wid == 0)
        def _():
            for r in range(8):
                for j in range(nj):
                    if r == 0:
                        accb = se[0] * eb_v[0, pl.ds(16 * j, 16)]
                        for e in range(1, _E):
                            accb = accb + se[e] * eb_v[e, pl.ds(16 * j, 16)]
                    else:
                        accb = jnp.zeros((16,), jnp.float32)
                    bb_v[r, pl.ds(16 * j, 16)] = accb
            pltpu.sync_copy(bb_v, cb_hbm.at[b])


def _conv_kernel(x1_ref, w_ref, b_ref, out_ref, *, th, width, chan, rb):
    i = pl.program_id(1)
    row0 = i * th
    center = _KW * (_KH // 2) + _KW // 2
    rr = jax.lax.broadcasted_iota(jnp.int32, (chan, chan), 0)
    cc = jax.lax.broadcasted_iota(jnp.int32, (chan, chan), 1)
    eye = (rr == cc).astype(jnp.float32)
    wf = width + 8
    m = th * wf
    dn = (((1,), (0,)), ((), ()))
    accs = [jnp.zeros((m, chan), jnp.float32) for _ in range(_KW)]
    for dy in range(_KH):
        slab = x1_ref[0, pl.ds(row0 + rb - 1 + dy, th), :, :]
        flat = slab.reshape(m, chan)
        for dx in range(_KW):
            tap = _KW * dy + dx
            wt = w_ref[0, pl.ds(tap * chan, chan), :]
            if tap == center:
                wt = wt + eye
            accs[dx] = accs[dx] + jax.lax.dot_general(
                flat, wt.astype(jnp.bfloat16), dn,
                preferred_element_type=jnp.float32)
    zrow = jnp.zeros((1, chan), jnp.float32)
    out = (accs[1]
           + jnp.concatenate([zrow, accs[0][:-1, :]], axis=0)
           + jnp.concatenate([accs[2][1:, :], zrow], axis=0))
    out = out.reshape(th, wf, chan)[:, 0:width, :] + b_ref[0]
    out_ref[...] = jnp.transpose(out, (2, 0, 1))[None]


def kernel(inputs, k, expert_w, expert_b, gate_w, gate_b):
    bsz, chan, height, width = inputs.shape
    n_pixels = height * width
    rb = 32
    n_rb = height // rb
    th = 32
    ni = height // th
    hp = height + 2 * rb
    wp = width + 8
    n_taps = _KH * _KW
    rows_per_w = 32

    ew3 = expert_w.transpose(0, 3, 4, 2, 1).reshape(_E, n_taps * chan, chan)
    gwt_flat = gate_w.T.reshape(_E * chan)
    gb16 = jnp.pad(gate_b, (0, 16 - _E))
    k16 = jnp.pad(k, (0, 15))

    x1, pooled = pl.pallas_call(
        functools.partial(_fmt_kernel, n_rb=n_rb),
        grid=(bsz, n_rb + 2),
        in_specs=[pl.BlockSpec(
            (1, chan, rb, width),
            lambda b, i: (b, 0, jnp.clip(i - 1, 0, n_rb - 1), 0))],
        out_specs=[
            pl.BlockSpec((1, rb, wp, chan), lambda b, i: (b, i, 0, 0)),
            pl.BlockSpec((1, 1, chan), lambda b, i: (b, 0, 0)),
        ],
        out_shape=[
            jax.ShapeDtypeStruct((bsz, hp, wp, chan), jnp.bfloat16),
            jax.ShapeDtypeStruct((bsz, 1, chan), jnp.float32),
        ],
    )(inputs)

    mesh = plsc.VectorSubcoreMesh(core_axis_name="c", subcore_axis_name="s")
    sc_fn = functools.partial(
        pl.kernel,
        mesh=mesh,
        out_type=[
            jax.ShapeDtypeStruct((bsz, n_taps * chan, chan), jnp.float32),
            jax.ShapeDtypeStruct((bsz, 8, chan), jnp.float32),
        ],
        scratch_types=[
            pltpu.VMEM((bsz * chan,), jnp.float32),
            pltpu.VMEM((_E * chan,), jnp.float32),
            pltpu.VMEM((16,), jnp.float32),
            pltpu.VMEM((16,), jnp.float32),
            pltpu.VMEM((_E, rows_per_w, chan), jnp.float32),
            pltpu.VMEM((rows_per_w, chan), jnp.float32),
            pltpu.VMEM((_E, chan), jnp.float32),
            pltpu.VMEM((8, chan), jnp.float32),
            pltpu.SemaphoreType.DMA,
        ],
    )(functools.partial(
        _sc_gate, bsz=bsz, chan=chan, n_pixels=n_pixels,
        rows_per_w=rows_per_w))
    cw, cb = sc_fn(pooled.reshape(bsz * chan), gwt_flat, gb16, ew3,
                   expert_b, k16)

    out = pl.pallas_call(
        functools.partial(_conv_kernel, th=th, width=width, chan=chan, rb=rb),
        grid=(bsz, ni),
        in_specs=[
            pl.BlockSpec((1, hp, wp, chan), lambda b, i: (b, 0, 0, 0)),
            pl.BlockSpec((1, n_taps * chan, chan), lambda b, i: (b, 0, 0)),
            pl.BlockSpec((1, 1, chan), lambda b, i: (b, 0, 0)),
        ],
        out_specs=pl.BlockSpec((1, chan, th, width), lambda b, i: (b, 0, i, 0)),
        out_shape=jax.ShapeDtypeStruct((bsz, chan, height, width), jnp.float32),
    )(x1, cw, cb[:, 0, :].reshape(bsz, 1, chan))

    return out


# speedup vs baseline: 1.0384x; 1.0384x over previous
import functools

import jax
import jax.numpy as jnp
from jax import lax
from jax.experimental import pallas as pl
from jax.experimental.pallas import tpu as pltpu
from jax.experimental.pallas import tpu_sc as plsc

_E = 8
_KH = _KW = 3


def _pool_kernel(x_ref, out_ref):
    i = pl.program_id(1)

    @pl.when(i == 0)
    def _():
        out_ref[...] = jnp.zeros_like(out_ref)

    out_ref[...] += jnp.sum(x_ref[...], axis=(2, 3))[:, None, :]


def _fmt_kernel(x_ref, x1_ref, *, n_rb):
    i = pl.program_id(1)

    @pl.when((i >= 1) & (i <= n_rb))
    def _():
        t = jnp.transpose(x_ref[0].astype(jnp.bfloat16), (1, 2, 0))
        x1_ref[0, :, 0 : t.shape[1], :] = t
        x1_ref[0, :, t.shape[1] :, :] = jnp.zeros_like(
            x1_ref[0, :, t.shape[1] :, :])

    @pl.when((i == 0) | (i == n_rb + 1))
    def _():
        x1_ref[...] = jnp.zeros_like(x1_ref)


def _sc_gate(pooled_hbm, gwt_hbm, gb_hbm, ew_hbm, eb_hbm, k_hbm,
             cw_hbm, cb_hbm,
             pooled_v, gw_v, gb_v, k_v, chunk_v, acc_v, eb_v, bb_v, sem,
             *, bsz, chan, n_pixels, rows_per_w):
    nc = 2
    wid = lax.axis_index("s") * nc + lax.axis_index("c")
    iota = lax.broadcasted_iota(jnp.int32, (16,), 0)

    gdn = lax.GatherDimensionNumbers(
        offset_dims=(), collapsed_slice_dims=(0,), start_index_map=(0,))

    def _tree(v, op):
        for sh in (8, 4, 2, 1):
            idx = jnp.bitwise_and(iota + sh, 15)
            rot = lax.gather(v, idx[:, None], gdn, slice_sizes=(1,),
                             mode=lax.GatherScatterMode.PROMISE_IN_BOUNDS)
            v = op(v, rot)
        return v[0]

    def _vsum(v):
        return _tree(v, jnp.add)

    def _vmax(v):
        return _tree(v, jnp.maximum)

    def _vmin(v):
        return _tree(v, jnp.minimum)

    pltpu.sync_copy(pooled_hbm, pooled_v)
    pltpu.sync_copy(gwt_hbm, gw_v)
    pltpu.sync_copy(gb_hbm, gb_v)
    pltpu.sync_copy(k_hbm, k_v)
    pltpu.sync_copy(eb_hbm, eb_v)
    kscal = k_v[...][0]
    gbv = gb_v[...]

    nj = chan // 16
    for b in range(bsz):
        lv = jnp.full((16,), -jnp.inf, jnp.float32)
        for e in range(_E):
            ve = jnp.zeros((16,), jnp.float32)
            for j in range(nj):
                ve = ve + (pooled_v[pl.ds(b * chan + 16 * j, 16)]
                           * gw_v[pl.ds(e * chan + 16 * j, 16)])
            logit = _vsum(ve) * (1.0 / n_pixels) + gbv[e]
            lv = jnp.where(iota == e, logit, lv)
        mask = iota < _E
        mx = _vmax(jnp.where(mask, lv, -jnp.inf))
        ex = jnp.where(mask, jnp.exp(lv - mx), 0.0)
        w = ex / _vsum(ex)
        m1 = _vmax(w)
        i1 = _vmin(jnp.where(w == m1, iota, 16))
        w2 = jnp.where(iota == i1, -1.0, w)
        m2 = _vmax(w2)
        i2 = _vmin(jnp.where(w2 == m2, iota, 16))
        scale = (jnp.where(iota == i1, m1, 0.0)
                 + jnp.where(iota == i2, m2, 0.0)) * kscal

        se = [scale[e] for e in range(_E)]
        n_active = (_KH * _KW * chan) // rows_per_w

        @pl.when(wid < n_active)
        def _():
            cps = []
            for e in range(_E):
                cps.append(pltpu.async_copy(
                    ew_hbm.at[e, pl.ds(wid * rows_per_w, rows_per_w), :],
                    chunk_v.at[e], sem))
            for cp in cps:
                cp.wait()

            def body(r, _):
                for j in range(nj):
                    acc = se[0] * chunk_v[0, r, pl.ds(16 * j, 16)]
                    for e in range(1, _E):
                        acc = acc + se[e] * chunk_v[e, r, pl.ds(16 * j, 16)]
                    acc_v[r, pl.ds(16 * j, 16)] = acc
                return 0

            lax.fori_loop(0, rows_per_w, body, 0)
            pltpu.sync_copy(
                acc_v, cw_hbm.at[b, pl.ds(wid * rows_per_w, rows_per_w), :])

        @pl.when(wid == 0)
        def _():
            for r in range(8):
                for j in range(nj):
                    if r == 0:
                        accb = se[0] * eb_v[0, pl.ds(16 * j, 16)]
                        for e in range(1, _E):
                            accb = accb + se[e] * eb_v[e, pl.ds(16 * j, 16)]
                    else:
                        accb = jnp.zeros((16,), jnp.float32)
                    bb_v[r, pl.ds(16 * j, 16)] = accb
            pltpu.sync_copy(bb_v, cb_hbm.at[b])


def _conv_kernel(x1_ref, w_ref, b_ref, out_ref, *, th, width, chan, rb):
    i = pl.program_id(1)
    row0 = i * th
    center = _KW * (_KH // 2) + _KW // 2
    rr = jax.lax.broadcasted_iota(jnp.int32, (chan, chan), 0)
    cc = jax.lax.broadcasted_iota(jnp.int32, (chan, chan), 1)
    eye = (rr == cc).astype(jnp.float32)
    wf = width + 8
    m = th * wf
    dn = (((1,), (0,)), ((), ()))
    accs = [jnp.zeros((m, chan), jnp.float32) for _ in range(_KW)]
    for dy in range(_KH):
        slab = x1_ref[0, pl.ds(row0 + rb - 1 + dy, th), :, :]
        flat = slab.reshape(m, chan)
        for dx in range(_KW):
            tap = _KW * dy + dx
            wt = w_ref[0, pl.ds(tap * chan, chan), :]
            if tap == center:
                wt = wt + eye
            accs[dx] = accs[dx] + jax.lax.dot_general(
                flat, wt.astype(jnp.bfloat16), dn,
                preferred_element_type=jnp.float32)
    zrow = jnp.zeros((1, chan), jnp.float32)
    out = (accs[1]
           + jnp.concatenate([zrow, accs[0][:-1, :]], axis=0)
           + jnp.concatenate([accs[2][1:, :], zrow], axis=0))
    out = out.reshape(th, wf, chan)[:, 0:width, :] + b_ref[0]
    out_ref[...] = jnp.transpose(out, (2, 0, 1))[None]


def kernel(inputs, k, expert_w, expert_b, gate_w, gate_b):
    bsz, chan, height, width = inputs.shape
    n_pixels = height * width
    rb = 32
    n_rb = height // rb
    th = 32
    ni = height // th
    hp = height + 2 * rb
    wp = width + 8
    n_taps = _KH * _KW
    rows_per_w = 32

    ew3 = expert_w.transpose(0, 3, 4, 2, 1).reshape(_E, n_taps * chan, chan)
    gwt_flat = gate_w.T.reshape(_E * chan)
    gb16 = jnp.pad(gate_b, (0, 16 - _E))
    k16 = jnp.pad(k, (0, 15))

    pooled = pl.pallas_call(
        _pool_kernel,
        grid=(bsz, 4),
        in_specs=[pl.BlockSpec((1, chan, height // 4, width),
                               lambda b, i: (b, 0, i, 0))],
        out_specs=pl.BlockSpec((1, 1, chan), lambda b, i: (b, 0, 0)),
        out_shape=jax.ShapeDtypeStruct((bsz, 1, chan), jnp.float32),
    )(inputs)

    x1 = pl.pallas_call(
        functools.partial(_fmt_kernel, n_rb=n_rb),
        grid=(bsz, n_rb + 2),
        in_specs=[pl.BlockSpec(
            (1, chan, rb, width),
            lambda b, i: (b, 0, jnp.clip(i - 1, 0, n_rb - 1), 0))],
        out_specs=pl.BlockSpec((1, rb, wp, chan), lambda b, i: (b, i, 0, 0)),
        out_shape=jax.ShapeDtypeStruct((bsz, hp, wp, chan), jnp.bfloat16),
    )(inputs)

    mesh = plsc.VectorSubcoreMesh(core_axis_name="c", subcore_axis_name="s")
    sc_fn = functools.partial(
        pl.kernel,
        mesh=mesh,
        out_type=[
            jax.ShapeDtypeStruct((bsz, n_taps * chan, chan), jnp.float32),
            jax.ShapeDtypeStruct((bsz, 8, chan), jnp.float32),
        ],
        scratch_types=[
            pltpu.VMEM((bsz * chan,), jnp.float32),
            pltpu.VMEM((_E * chan,), jnp.float32),
            pltpu.VMEM((16,), jnp.float32),
            pltpu.VMEM((16,), jnp.float32),
            pltpu.VMEM((_E, rows_per_w, chan), jnp.float32),
            pltpu.VMEM((rows_per_w, chan), jnp.float32),
            pltpu.VMEM((_E, chan), jnp.float32),
            pltpu.VMEM((8, chan), jnp.float32),
            pltpu.SemaphoreType.DMA,
        ],
    )(functools.partial(
        _sc_gate, bsz=bsz, chan=chan, n_pixels=n_pixels,
        rows_per_w=rows_per_w))
    cw, cb = sc_fn(pooled.reshape(bsz * chan), gwt_flat, gb16, ew3,
                   expert_b, k16)

    out = pl.pallas_call(
        functools.partial(_conv_kernel, th=th, width=width, chan=chan, rb=rb),
        grid=(bsz, ni),
        in_specs=[
            pl.BlockSpec((1, hp, wp, chan), lambda b, i: (b, 0, 0, 0)),
            pl.BlockSpec((1, n_taps * chan, chan), lambda b, i: (b, 0, 0)),
            pl.BlockSpec((1, 1, chan), lambda b, i: (b, 0, 0)),
        ],
        out_specs=pl.BlockSpec((1, chan, th, width), lambda b, i: (b, 0, i, 0)),
        out_shape=jax.ShapeDtypeStruct((bsz, chan, height, width), jnp.float32),
    )(x1, cw, cb[:, 0, :].reshape(bsz, 1, chan))

    return out
